# initial kernel scaffold (unmeasured)
import jax
import jax.numpy as jnp
from jax import lax
from jax.experimental import pallas as pl
from jax.experimental.pallas import tpu as pltpu

N_DEV = 4
M_PER = 1024
K = 4096
N = 8192
N_PER = N // N_DEV
KBLK = 1024
N_KBLK = N // KBLK
QCHUNK = 256


def kernel(x, w_mat):
    def body(x_ref, w_hbm, out_hbm,
             wbuf, y_ref, amax_ref, sendq, recvq, stage,
             copy_sems, amax_send_sems, amax_recv_sems,
             dsend_sems, drecv_sems, out_sem):
        me = lax.axis_index("i")

        def wcopy(j, slot):
            return pltpu.make_async_copy(
                w_hbm.at[:, pl.ds(j * KBLK, KBLK)],
                wbuf.at[slot],
                copy_sems.at[slot],
            )

        wcopy(0, 0).start()
        amax = jnp.float32(0.0)
        for j in range(N_KBLK):
            slot = j % 2
            if j + 1 < N_KBLK:
                wcopy(j + 1, (j + 1) % 2).start()
            wcopy(j, slot).wait()
            y = jnp.dot(x_ref[...], wbuf[slot],
                        preferred_element_type=jnp.float32)
            y = jnp.maximum(y, 0.0)
            amax = jnp.maximum(amax, jnp.max(y))
            y_ref[:, pl.ds(j * KBLK, KBLK)] = y.astype(jnp.bfloat16)

        barrier = pltpu.get_barrier_semaphore()
        for k in range(1, N_DEV):
            pl.semaphore_signal(
                barrier, inc=1,
                device_id=((me + k) % N_DEV,),
                device_id_type=pl.DeviceIdType.MESH,
            )
        pl.semaphore_wait(barrier, N_DEV - 1)

        amax_ref[pl.ds(0, 1), :] = jnp.full((1, 128), amax, jnp.float32)
        amax_ref[pl.ds(me, 1), :] = jnp.full((1, 128), amax, jnp.float32)
        amax_rdmas = []
        for k in range(1, N_DEV):
            dst = (me + k) % N_DEV
            rdma = pltpu.make_async_remote_copy(
                src_ref=amax_ref.at[pl.ds(me, 1)],
                dst_ref=amax_ref.at[pl.ds(me, 1)],
                send_sem=amax_send_sems.at[k],
                recv_sem=amax_recv_sems.at[k],
                device_id=(dst,),
                device_id_type=pl.DeviceIdType.MESH,
            )
            rdma.start()
            amax_rdmas.append(rdma)
        for k in range(1, N_DEV):
            src_chip = (me + N_DEV - k) % N_DEV
            recv = pltpu.make_async_remote_copy(
                src_ref=amax_ref.at[pl.ds(me, 1)],
                dst_ref=amax_ref.at[pl.ds(src_chip, 1)],
                send_sem=amax_send_sems.at[0],
                recv_sem=amax_recv_sems.at[k],
                device_id=(src_chip,),
                device_id_type=pl.DeviceIdType.MESH,
            )
            recv.wait_recv()
        gamax = jnp.max(amax_ref[...])
        scale = gamax / 448.0
        inv_scale = 448.0 / gamax

        data_rdmas = []
        for k in range(1, N_DEV):
            dst = (me + k) % N_DEV
            for c in range(M_PER // QCHUNK):
                yblk = y_ref[pl.ds(c * QCHUNK, QCHUNK),
                             pl.ds(dst * N_PER, N_PER)]
                q = jnp.minimum(yblk.astype(jnp.float32) * inv_scale, 448.0)
                sendq[k - 1, pl.ds(c * QCHUNK, QCHUNK), :] = q.astype(
                    jnp.float8_e4m3fn)
            rdma = pltpu.make_async_remote_copy(
                src_ref=sendq.at[k - 1],
                dst_ref=recvq.at[me],
                send_sem=dsend_sems.at[k],
                recv_sem=drecv_sems.at[k],
                device_id=(dst,),
                device_id_type=pl.DeviceIdType.MESH,
            )
            rdma.start()
            data_rdmas.append(rdma)

        for c in range(M_PER // QCHUNK):
            yblk = y_ref[pl.ds(c * QCHUNK, QCHUNK), pl.ds(me * N_PER, N_PER)]
            q = jnp.minimum(yblk.astype(jnp.float32) * inv_scale, 448.0)
            q = q.astype(jnp.float8_e4m3fn)
            stage[pl.ds(c * QCHUNK, QCHUNK), :] = (
                q.astype(jnp.float32) * scale).astype(jnp.bfloat16)
        own_copy = pltpu.make_async_copy(
            stage, out_hbm.at[pl.ds(me * M_PER, M_PER)], out_sem)
        own_copy.start()
        own_copy.wait()

        for k in range(1, N_DEV):
            src_chip = (me + N_DEV - k) % N_DEV
            recv = pltpu.make_async_remote_copy(
                src_ref=sendq.at[0],
                dst_ref=recvq.at[src_chip],
                send_sem=dsend_sems.at[0],
                recv_sem=drecv_sems.at[k],
                device_id=(src_chip,),
                device_id_type=pl.DeviceIdType.MESH,
            )
            recv.wait_recv()
            for c in range(M_PER // QCHUNK):
                q = recvq[src_chip, pl.ds(c * QCHUNK, QCHUNK), :]
                stage[pl.ds(c * QCHUNK, QCHUNK), :] = (
                    q.astype(jnp.float32) * scale).astype(jnp.bfloat16)
            cp = pltpu.make_async_copy(
                stage, out_hbm.at[pl.ds(src_chip * M_PER, M_PER)], out_sem)
            cp.start()
            cp.wait()

        for rdma in amax_rdmas + data_rdmas:
            rdma.wait_send()

    return pl.pallas_call(
        body,
        out_shape=jax.ShapeDtypeStruct((N_DEV * M_PER, N_PER), jnp.bfloat16),
        in_specs=[
            pl.BlockSpec(memory_space=pltpu.VMEM),
            pl.BlockSpec(memory_space=pltpu.ANY),
        ],
        out_specs=pl.BlockSpec(memory_space=pltpu.ANY),
        scratch_shapes=[
            pltpu.VMEM((2, K, KBLK), jnp.bfloat16),
            pltpu.VMEM((M_PER, N), jnp.bfloat16),
            pltpu.VMEM((N_DEV, 128), jnp.float32),
            pltpu.VMEM((N_DEV - 1, M_PER, N_PER), jnp.float8_e4m3fn),
            pltpu.VMEM((N_DEV, M_PER, N_PER), jnp.float8_e4m3fn),
            pltpu.VMEM((M_PER, N_PER), jnp.bfloat16),
            pltpu.SemaphoreType.DMA((2,)),
            pltpu.SemaphoreType.DMA((N_DEV,)),
            pltpu.SemaphoreType.DMA((N_DEV,)),
            pltpu.SemaphoreType.DMA((N_DEV,)),
            pltpu.SemaphoreType.DMA((N_DEV,)),
            pltpu.SemaphoreType.DMA,
        ],
        compiler_params=pltpu.CompilerParams(collective_id=0),
    )(x, w_mat)


# baseline (device time: 235735 ns/iter reference)
import jax
import jax.numpy as jnp
from jax import lax
from jax.experimental import pallas as pl
from jax.experimental.pallas import tpu as pltpu

N_DEV = 4
M_PER = 1024
K = 4096
N = 8192
N_PER = N // N_DEV
KBLK = 256
N_KBLK = N // KBLK
QCHUNK = 256


def kernel(x, w_mat):
    def body(x_ref, w_hbm, out_hbm,
             wbuf, y_ref, amax_ref, sendq, recvq, stage,
             copy_sems, amax_send_sems, amax_recv_sems,
             dsend_sems, drecv_sems, out_sem):
        me = lax.axis_index("i")

        def wcopy(j, slot):
            return pltpu.make_async_copy(
                w_hbm.at[:, pl.ds(j * KBLK, KBLK)],
                wbuf.at[slot],
                copy_sems.at[slot],
            )

        wcopy(0, 0).start()
        amax = jnp.float32(0.0)
        for j in range(N_KBLK):
            slot = j % 2
            if j + 1 < N_KBLK:
                wcopy(j + 1, (j + 1) % 2).start()
            wcopy(j, slot).wait()
            y = jnp.dot(x_ref[...], wbuf[slot],
                        preferred_element_type=jnp.float32)
            y = jnp.maximum(y, 0.0)
            amax = jnp.maximum(amax, jnp.max(y))
            y_ref[:, pl.ds(j * KBLK, KBLK)] = y.astype(jnp.bfloat16)

        barrier = pltpu.get_barrier_semaphore()
        for k in range(1, N_DEV):
            pl.semaphore_signal(
                barrier, inc=1,
                device_id=((me + k) % N_DEV,),
                device_id_type=pl.DeviceIdType.MESH,
            )
        pl.semaphore_wait(barrier, N_DEV - 1)

        amax_ref[pl.ds(me, 1), :] = jnp.full((1, 128), amax, jnp.float32)
        amax_rdmas = []
        for k in range(1, N_DEV):
            dst = (me + k) % N_DEV
            rdma = pltpu.make_async_remote_copy(
                src_ref=amax_ref.at[pl.ds(me, 1)],
                dst_ref=amax_ref.at[pl.ds(me, 1)],
                send_sem=amax_send_sems.at[k],
                recv_sem=amax_recv_sems.at[k],
                device_id=(dst,),
                device_id_type=pl.DeviceIdType.MESH,
            )
            rdma.start()
            amax_rdmas.append(rdma)
        for k in range(1, N_DEV):
            src_chip = (me + N_DEV - k) % N_DEV
            recv = pltpu.make_async_remote_copy(
                src_ref=amax_ref.at[pl.ds(me, 1)],
                dst_ref=amax_ref.at[pl.ds(src_chip, 1)],
                send_sem=amax_send_sems.at[0],
                recv_sem=amax_recv_sems.at[k],
                device_id=(src_chip,),
                device_id_type=pl.DeviceIdType.MESH,
            )
            recv.wait_recv()
        gamax = jnp.max(amax_ref[...])
        scale = gamax / 448.0
        inv_scale = 448.0 / gamax

        data_rdmas = []
        for k in range(1, N_DEV):
            dst = (me + k) % N_DEV
            for c in range(M_PER // QCHUNK):
                yblk = y_ref[pl.ds(c * QCHUNK, QCHUNK),
                             pl.ds(dst * N_PER, N_PER)]
                q = jnp.minimum(yblk.astype(jnp.float32) * inv_scale, 448.0)
                sendq[k - 1, pl.ds(c * QCHUNK, QCHUNK), :] = q.astype(
                    jnp.float8_e4m3fn)
            rdma = pltpu.make_async_remote_copy(
                src_ref=sendq.at[k - 1],
                dst_ref=recvq.at[k - 1],
                send_sem=dsend_sems.at[k],
                recv_sem=drecv_sems.at[k],
                device_id=(dst,),
                device_id_type=pl.DeviceIdType.MESH,
            )
            rdma.start()
            data_rdmas.append(rdma)

        for c in range(M_PER // QCHUNK):
            yblk = y_ref[pl.ds(c * QCHUNK, QCHUNK), pl.ds(me * N_PER, N_PER)]
            q = jnp.minimum(yblk.astype(jnp.float32) * inv_scale, 448.0)
            q = q.astype(jnp.float8_e4m3fn)
            stage[...] = (q.astype(jnp.float32) * scale).astype(jnp.bfloat16)
            cp = pltpu.make_async_copy(
                stage,
                out_hbm.at[pl.ds(me * M_PER + c * QCHUNK, QCHUNK)],
                out_sem)
            cp.start()
            cp.wait()

        for k in range(1, N_DEV):
            src_chip = (me + N_DEV - k) % N_DEV
            recv = pltpu.make_async_remote_copy(
                src_ref=sendq.at[0],
                dst_ref=recvq.at[k - 1],
                send_sem=dsend_sems.at[0],
                recv_sem=drecv_sems.at[k],
                device_id=(src_chip,),
                device_id_type=pl.DeviceIdType.MESH,
            )
            recv.wait_recv()
            for c in range(M_PER // QCHUNK):
                q = recvq[k - 1, pl.ds(c * QCHUNK, QCHUNK), :]
                stage[...] = (q.astype(jnp.float32) * scale).astype(
                    jnp.bfloat16)
                cp = pltpu.make_async_copy(
                    stage,
                    out_hbm.at[pl.ds(src_chip * M_PER + c * QCHUNK, QCHUNK)],
                    out_sem)
                cp.start()
                cp.wait()

        for rdma in amax_rdmas + data_rdmas:
            rdma.wait_send()

    return pl.pallas_call(
        body,
        out_shape=jax.ShapeDtypeStruct((N_DEV * M_PER, N_PER), jnp.bfloat16),
        in_specs=[
            pl.BlockSpec(memory_space=pltpu.MemorySpace.VMEM),
            pl.BlockSpec(memory_space=pl.ANY),
        ],
        out_specs=pl.BlockSpec(memory_space=pl.ANY),
        scratch_shapes=[
            pltpu.VMEM((2, K, KBLK), jnp.float32),
            pltpu.VMEM((M_PER, N), jnp.bfloat16),
            pltpu.VMEM((N_DEV, 128), jnp.float32),
            pltpu.VMEM((N_DEV - 1, M_PER, N_PER), jnp.float8_e4m3fn),
            pltpu.VMEM((N_DEV - 1, M_PER, N_PER), jnp.float8_e4m3fn),
            pltpu.VMEM((QCHUNK, N_PER), jnp.bfloat16),
            pltpu.SemaphoreType.DMA((2,)),
            pltpu.SemaphoreType.DMA((N_DEV,)),
            pltpu.SemaphoreType.DMA((N_DEV,)),
            pltpu.SemaphoreType.DMA((N_DEV,)),
            pltpu.SemaphoreType.DMA((N_DEV,)),
            pltpu.SemaphoreType.DMA,
        ],
        compiler_params=pltpu.CompilerParams(
            collective_id=0, vmem_limit_bytes=100 * 1024 * 1024),
    )(x, w_mat)


# device time: 225613 ns/iter; 1.0449x vs baseline; 1.0449x over previous
import jax
import jax.numpy as jnp
from jax import lax
from jax.experimental import pallas as pl
from jax.experimental.pallas import tpu as pltpu

N_DEV = 4
M_PER = 1024
K = 4096
N = 8192
N_PER = N // N_DEV
KBLK = 256
N_KBLK = N // KBLK
NSPLIT = 4
QCHUNK = 256
NCHUNK = M_PER // QCHUNK

SEND_ORDER = (2, 1, 3)
RECV_ORDER = (1, 3, 2)


def kernel(x, w_mat):
    def body(x_ref, w_hbm, out_hbm,
             wbuf, y_ref, amax_ref, sendq, recvq, stage,
             copy_sems, amax_send_sems, amax_recv_sems,
             dsend_sems, drecv_sems, out_sems):
        me = lax.axis_index("i")

        RQ = K // NSPLIT

        def wcopies(j, slot):
            return [
                pltpu.make_async_copy(
                    w_hbm.at[pl.ds(h * RQ, RQ), pl.ds(j * KBLK, KBLK)],
                    wbuf.at[slot, pl.ds(h * RQ, RQ)],
                    copy_sems.at[slot, h],
                )
                for h in range(NSPLIT)
            ]

        for cp in wcopies(0, 0):
            cp.start()
        amax = jnp.float32(0.0)
        y_prev = None
        for j in range(N_KBLK):
            slot = j % 2
            if j + 1 < N_KBLK:
                for cp in wcopies(j + 1, (j + 1) % 2):
                    cp.start()
            for cp in wcopies(j, slot):
                cp.wait()
            y = jnp.dot(x_ref[...], wbuf[slot],
                        preferred_element_type=jnp.float32)
            if y_prev is not None:
                amax = jnp.maximum(amax, jnp.max(y_prev))
                y_ref[:, pl.ds((j - 1) * KBLK, KBLK)] = y_prev.astype(
                    jnp.bfloat16)
            y_prev = y
        amax = jnp.maximum(amax, jnp.max(y_prev))
        y_ref[:, pl.ds((N_KBLK - 1) * KBLK, KBLK)] = y_prev.astype(
            jnp.bfloat16)

        barrier = pltpu.get_barrier_semaphore()
        for k in range(1, N_DEV):
            pl.semaphore_signal(
                barrier, inc=1,
                device_id=((me + k) % N_DEV,),
                device_id_type=pl.DeviceIdType.MESH,
            )
        pl.semaphore_wait(barrier, N_DEV - 1)

        amax_ref[pl.ds(me, 1), :] = jnp.full((1, 128), amax, jnp.float32)
        amax_rdmas = []
        for k in range(1, N_DEV):
            dst = (me + k) % N_DEV
            rdma = pltpu.make_async_remote_copy(
                src_ref=amax_ref.at[pl.ds(me, 1)],
                dst_ref=amax_ref.at[pl.ds(me, 1)],
                send_sem=amax_send_sems.at[k],
                recv_sem=amax_recv_sems.at[k],
                device_id=(dst,),
                device_id_type=pl.DeviceIdType.MESH,
            )
            rdma.start()
            amax_rdmas.append(rdma)
        for k in range(1, N_DEV):
            src_chip = (me + N_DEV - k) % N_DEV
            recv = pltpu.make_async_remote_copy(
                src_ref=amax_ref.at[pl.ds(me, 1)],
                dst_ref=amax_ref.at[pl.ds(src_chip, 1)],
                send_sem=amax_send_sems.at[0],
                recv_sem=amax_recv_sems.at[k],
                device_id=(src_chip,),
                device_id_type=pl.DeviceIdType.MESH,
            )
            recv.wait_recv()
        gamax = jnp.max(amax_ref[...])
        scale = gamax / 448.0
        inv_scale = 448.0 / gamax
        scale_bf = scale.astype(jnp.bfloat16)

        data_rdmas = []
        for c in range(NCHUNK):
            for k in SEND_ORDER:
                dst = (me + k) % N_DEV
                yblk = y_ref[pl.ds(c * QCHUNK, QCHUNK),
                             pl.ds(dst * N_PER, N_PER)]
                q = jnp.clip(yblk.astype(jnp.float32) * inv_scale,
                             0.0, 448.0)
                sendq[k - 1, pl.ds(c * QCHUNK, QCHUNK), :] = q.astype(
                    jnp.float8_e4m3fn)
                rdma = pltpu.make_async_remote_copy(
                    src_ref=sendq.at[k - 1, pl.ds(c * QCHUNK, QCHUNK)],
                    dst_ref=recvq.at[k - 1, pl.ds(c * QCHUNK, QCHUNK)],
                    send_sem=dsend_sems.at[k - 1, c],
                    recv_sem=drecv_sems.at[k - 1, c],
                    device_id=(dst,),
                    device_id_type=pl.DeviceIdType.MESH,
                )
                rdma.start()
                data_rdmas.append(rdma)

        pending = [None, None]
        slot_i = [0]

        def store_chunk(out_rows, qvals):
            sslot = slot_i[0]
            slot_i[0] ^= 1
            if pending[sslot] is not None:
                pltpu.make_async_copy(
                    stage.at[sslot],
                    out_hbm.at[pl.ds(pending[sslot], QCHUNK)],
                    out_sems.at[sslot]).wait()
            stage[sslot] = qvals.astype(jnp.bfloat16) * scale_bf
            pltpu.make_async_copy(
                stage.at[sslot],
                out_hbm.at[pl.ds(out_rows, QCHUNK)],
                out_sems.at[sslot]).start()
            pending[sslot] = out_rows

        def drain_stores():
            for sslot in (0, 1):
                if pending[sslot] is not None:
                    pltpu.make_async_copy(
                        stage.at[sslot],
                        out_hbm.at[pl.ds(pending[sslot], QCHUNK)],
                        out_sems.at[sslot]).wait()
                    pending[sslot] = None

        for c in range(NCHUNK):
            yblk = y_ref[pl.ds(c * QCHUNK, QCHUNK), pl.ds(me * N_PER, N_PER)]
            q = jnp.clip(yblk.astype(jnp.float32) * inv_scale, 0.0, 448.0)
            store_chunk(me * M_PER + c * QCHUNK,
                        q.astype(jnp.float8_e4m3fn))

        for c in range(NCHUNK):
            for k in RECV_ORDER:
                src_chip = (me + N_DEV - k) % N_DEV
                recv = pltpu.make_async_remote_copy(
                    src_ref=sendq.at[k - 1, pl.ds(c * QCHUNK, QCHUNK)],
                    dst_ref=recvq.at[k - 1, pl.ds(c * QCHUNK, QCHUNK)],
                    send_sem=dsend_sems.at[k - 1, c],
                    recv_sem=drecv_sems.at[k - 1, c],
                    device_id=(src_chip,),
                    device_id_type=pl.DeviceIdType.MESH,
                )
                recv.wait_recv()
                store_chunk(src_chip * M_PER + c * QCHUNK,
                            recvq[k - 1, pl.ds(c * QCHUNK, QCHUNK), :])
        drain_stores()

        for rdma in amax_rdmas + data_rdmas:
            rdma.wait_send()

    return pl.pallas_call(
        body,
        out_shape=jax.ShapeDtypeStruct((N_DEV * M_PER, N_PER), jnp.bfloat16),
        in_specs=[
            pl.BlockSpec(memory_space=pltpu.MemorySpace.VMEM),
            pl.BlockSpec(memory_space=pl.ANY),
        ],
        out_specs=pl.BlockSpec(memory_space=pl.ANY),
        scratch_shapes=[
            pltpu.VMEM((2, K, KBLK), jnp.float32),
            pltpu.VMEM((M_PER, N), jnp.bfloat16),
            pltpu.VMEM((N_DEV, 128), jnp.float32),
            pltpu.VMEM((N_DEV - 1, M_PER, N_PER), jnp.float8_e4m3fn),
            pltpu.VMEM((N_DEV - 1, M_PER, N_PER), jnp.float8_e4m3fn),
            pltpu.VMEM((2, QCHUNK, N_PER), jnp.bfloat16),
            pltpu.SemaphoreType.DMA((2, NSPLIT)),
            pltpu.SemaphoreType.DMA((N_DEV,)),
            pltpu.SemaphoreType.DMA((N_DEV,)),
            pltpu.SemaphoreType.DMA((N_DEV - 1, NCHUNK)),
            pltpu.SemaphoreType.DMA((N_DEV - 1, NCHUNK)),
            pltpu.SemaphoreType.DMA((2,)),
        ],
        compiler_params=pltpu.CompilerParams(
            collective_id=0, vmem_limit_bytes=100 * 1024 * 1024),
    )(x, w_mat)
